# baseline (speedup 1.0000x reference)
import functools

import jax
import jax.numpy as jnp
from jax import lax
from jax.experimental import pallas as pl
from jax.experimental.pallas import tpu as pltpu
from jax.experimental.pallas import tpu_sc as plsc

LANES = 16
NUM_CORES = 2
NUM_SUBCORES = 16
NUM_WORKERS = NUM_CORES * NUM_SUBCORES
CHUNK = 64
N_SLOT = 2


def _tec_body(heads_hbm, rel_hbm, tails_hbm, entity_hbm, relw_hbm, out_hbm,
              idx_h, idx_r, idx_t, rel_sh, hrbufs, tbufs, obufs,
              hsems, tsems, rsems, osems):
    batch = out_hbm.shape[0]
    embed = out_hbm.shape[1]
    b_per_w = batch // NUM_WORKERS
    n_chunks = b_per_w // CHUNK
    groups = n_chunks // N_SLOT
    sid = lax.axis_index("s")
    wid = sid * NUM_CORES + lax.axis_index("c")
    base = wid * b_per_w

    pltpu.sync_copy(heads_hbm.at[pl.ds(base, b_per_w)], idx_h)
    pltpu.sync_copy(rel_hbm.at[pl.ds(base, b_per_w)], idx_r)
    pltpu.sync_copy(tails_hbm.at[pl.ds(base, b_per_w)], idx_t)

    def h_copy(c, k):
        s = pl.ds(c * CHUNK, CHUNK)
        return pltpu.make_async_copy(entity_hbm.at[idx_h.at[s]], hrbufs[k],
                                     hsems[k])

    def t_copy(c, k):
        s = pl.ds(c * CHUNK, CHUNK)
        return pltpu.make_async_copy(entity_hbm.at[idx_t.at[s]], tbufs[k],
                                     tsems[k])

    def r_add(c, k):
        s = pl.ds(c * CHUNK, CHUNK)
        return pltpu.async_copy(rel_sh.at[idx_r.at[s]], hrbufs[k], rsems[k],
                                add=True)

    def r_wait(k):
        pltpu.make_async_copy(rel_sh.at[pl.ds(0, CHUNK)], hrbufs[k],
                              rsems[k]).wait()

    def o_copy(c, k):
        return pltpu.make_async_copy(
            obufs[k], out_hbm.at[pl.ds(base + c * CHUNK, CHUNK)], osems[k])

    h_copy(0, 0).start()
    t_copy(0, 0).start()
    h_copy(1, 1).start()
    t_copy(1, 1).start()

    @pl.when(sid == 0)
    def _stage_rel():
        pltpu.sync_copy(relw_hbm, rel_sh)
    plsc.subcore_barrier()

    h_copy(0, 0).wait()
    r_add(0, 0)

    def group_body(g, carry):
        for k in range(N_SLOT):
            c = g * N_SLOT + k
            kn = (k + 1) % N_SLOT

            @pl.when(c + 1 < n_chunks)
            def _prime_next_radd():
                h_copy(c + 1, kn).wait()
                r_add(c + 1, kn)

            r_wait(k)
            t_copy(c, k).wait()

            @pl.when(g > 0)
            def _reclaim_obuf():
                o_copy(c, k).wait()

            hr = hrbufs[k]
            t = tbufs[k]
            o = obufs[k]

            @plsc.parallel_loop(0, CHUNK, 1, unroll=2)
            def _compute(j):
                for kk in range(embed // LANES):
                    s = pl.ds(kk * LANES, LANES)
                    o[j, s] = hr[j, s] - t[j, s]

            o_copy(c, k).start()

            @pl.when(g + 1 < groups)
            def _prefetch():
                h_copy(c + N_SLOT, k).start()
                t_copy(c + N_SLOT, k).start()
        return carry

    lax.fori_loop(0, groups, group_body, 0)

    for k in range(N_SLOT):
        o_copy(n_chunks - N_SLOT + k, k).wait()


def _body_wrapper(heads_hbm, rel_hbm, tails_hbm, entity_hbm, relw_hbm,
                  out_hbm, idx_h, idx_r, idx_t, *rest):
    n = N_SLOT
    rel_sh = rest[-1]
    rest = rest[:-1]
    hrbufs = tuple(rest[0:n])
    tbufs = tuple(rest[n:2 * n])
    obufs = tuple(rest[2 * n:3 * n])
    hsems = tuple(rest[3 * n:4 * n])
    tsems = tuple(rest[4 * n:5 * n])
    rsems = tuple(rest[5 * n:6 * n])
    osems = tuple(rest[6 * n:7 * n])
    _tec_body(heads_hbm, rel_hbm, tails_hbm, entity_hbm, relw_hbm, out_hbm,
              idx_h, idx_r, idx_t, rel_sh, hrbufs, tbufs, obufs,
              hsems, tsems, rsems, osems)


def kernel(heads, relations, tails, entity_weight, rel_weight):
    batch = heads.shape[0]
    embed = entity_weight.shape[1]
    b_per_w = batch // NUM_WORKERS
    heads = heads.astype(jnp.int32)
    relations = relations.astype(jnp.int32)
    tails = tails.astype(jnp.int32)

    mesh = plsc.VectorSubcoreMesh(core_axis_name="c", subcore_axis_name="s")
    scratch = [pltpu.VMEM((b_per_w,), jnp.int32)] * 3
    scratch += [pltpu.VMEM((CHUNK, embed), jnp.float32)] * (3 * N_SLOT)
    scratch += [pltpu.SemaphoreType.DMA] * (4 * N_SLOT)
    scratch += [pltpu.VMEM_SHARED(rel_weight.shape, jnp.float32)]
    run = functools.partial(
        pl.kernel,
        mesh=mesh,
        out_type=jax.ShapeDtypeStruct((batch, embed), jnp.float32),
        scratch_types=scratch,
    )(_body_wrapper)
    return run(heads, relations, tails, entity_weight, rel_weight)


# speedup vs baseline: 1.1152x; 1.1152x over previous
import functools

import jax
import jax.numpy as jnp
from jax import lax
from jax.experimental import pallas as pl
from jax.experimental.pallas import tpu as pltpu
from jax.experimental.pallas import tpu_sc as plsc

LANES = 16
NUM_CORES = 2
NUM_SUBCORES = 16
NUM_WORKERS = NUM_CORES * NUM_SUBCORES
CHUNK = 64
N_SLOT = 4


def _tec_body(heads_hbm, rel_hbm, tails_hbm, entity_hbm, relw_hbm, out_hbm,
              idx_h, idx_r, idx_t, rel_sh, hrbufs, tbufs, obufs,
              hsems, tsems, rsems, osems):
    batch = out_hbm.shape[0]
    embed = out_hbm.shape[1]
    b_per_w = batch // NUM_WORKERS
    n_chunks = b_per_w // CHUNK
    groups = n_chunks // N_SLOT
    sid = lax.axis_index("s")
    wid = sid * NUM_CORES + lax.axis_index("c")
    base = wid * b_per_w

    pltpu.sync_copy(heads_hbm.at[pl.ds(base, b_per_w)], idx_h)
    pltpu.sync_copy(rel_hbm.at[pl.ds(base, b_per_w)], idx_r)
    pltpu.sync_copy(tails_hbm.at[pl.ds(base, b_per_w)], idx_t)

    def h_copy(c, k):
        s = pl.ds(c * CHUNK, CHUNK)
        return pltpu.make_async_copy(entity_hbm.at[idx_h.at[s]], hrbufs[k],
                                     hsems[k])

    def t_copy(c, k):
        s = pl.ds(c * CHUNK, CHUNK)
        return pltpu.make_async_copy(entity_hbm.at[idx_t.at[s]], tbufs[k],
                                     tsems[k])

    def r_add(c, k):
        s = pl.ds(c * CHUNK, CHUNK)
        return pltpu.async_copy(rel_sh.at[idx_r.at[s]], hrbufs[k], rsems[k],
                                add=True)

    def r_wait(k):
        pltpu.make_async_copy(rel_sh.at[pl.ds(0, CHUNK)], hrbufs[k],
                              rsems[k]).wait()

    def o_copy(c, k):
        return pltpu.make_async_copy(
            obufs[k], out_hbm.at[pl.ds(base + c * CHUNK, CHUNK)], osems[k])

    for k in range(N_SLOT):
        h_copy(k, k).start()
        t_copy(k, k).start()

    @pl.when(sid == 0)
    def _stage_rel():
        pltpu.sync_copy(relw_hbm, rel_sh)
    plsc.subcore_barrier()

    h_copy(0, 0).wait()
    r_add(0, 0)

    def group_body(g, carry):
        for k in range(N_SLOT):
            c = g * N_SLOT + k
            kn = (k + 1) % N_SLOT

            @pl.when(c + 1 < n_chunks)
            def _prime_next_radd():
                h_copy(c + 1, kn).wait()
                r_add(c + 1, kn)

            r_wait(k)
            t_copy(c, k).wait()

            @pl.when(g > 0)
            def _reclaim_obuf():
                o_copy(c, k).wait()

            hr = hrbufs[k]
            t = tbufs[k]
            o = obufs[k]

            @plsc.parallel_loop(0, CHUNK, 1, unroll=2)
            def _compute(j):
                for kk in range(embed // LANES):
                    s = pl.ds(kk * LANES, LANES)
                    o[j, s] = hr[j, s] - t[j, s]

            o_copy(c, k).start()

            @pl.when(g + 1 < groups)
            def _prefetch():
                h_copy(c + N_SLOT, k).start()
                t_copy(c + N_SLOT, k).start()
        return carry

    lax.fori_loop(0, groups, group_body, 0)

    for k in range(N_SLOT):
        o_copy(n_chunks - N_SLOT + k, k).wait()


def _body_wrapper(heads_hbm, rel_hbm, tails_hbm, entity_hbm, relw_hbm,
                  out_hbm, idx_h, idx_r, idx_t, *rest):
    n = N_SLOT
    rel_sh = rest[-1]
    rest = rest[:-1]
    hrbufs = tuple(rest[0:n])
    tbufs = tuple(rest[n:2 * n])
    obufs = tuple(rest[2 * n:3 * n])
    hsems = tuple(rest[3 * n:4 * n])
    tsems = tuple(rest[4 * n:5 * n])
    rsems = tuple(rest[5 * n:6 * n])
    osems = tuple(rest[6 * n:7 * n])
    _tec_body(heads_hbm, rel_hbm, tails_hbm, entity_hbm, relw_hbm, out_hbm,
              idx_h, idx_r, idx_t, rel_sh, hrbufs, tbufs, obufs,
              hsems, tsems, rsems, osems)


def kernel(heads, relations, tails, entity_weight, rel_weight):
    batch = heads.shape[0]
    embed = entity_weight.shape[1]
    b_per_w = batch // NUM_WORKERS
    heads = heads.astype(jnp.int32)
    relations = relations.astype(jnp.int32)
    tails = tails.astype(jnp.int32)

    mesh = plsc.VectorSubcoreMesh(core_axis_name="c", subcore_axis_name="s")
    scratch = [pltpu.VMEM((b_per_w,), jnp.int32)] * 3
    scratch += [pltpu.VMEM((CHUNK, embed), jnp.float32)] * (3 * N_SLOT)
    scratch += [pltpu.SemaphoreType.DMA] * (4 * N_SLOT)
    scratch += [pltpu.VMEM_SHARED(rel_weight.shape, jnp.float32)]
    run = functools.partial(
        pl.kernel,
        mesh=mesh,
        out_type=jax.ShapeDtypeStruct((batch, embed), jnp.float32),
        scratch_types=scratch,
    )(_body_wrapper)
    return run(heads, relations, tails, entity_weight, rel_weight)


# speedup vs baseline: 1.1179x; 1.0024x over previous
import functools

import jax
import jax.numpy as jnp
from jax import lax
from jax.experimental import pallas as pl
from jax.experimental.pallas import tpu as pltpu
from jax.experimental.pallas import tpu_sc as plsc

LANES = 16
NUM_CORES = 2
NUM_SUBCORES = 16
NUM_WORKERS = NUM_CORES * NUM_SUBCORES
CHUNK = 64
N_SLOT = 4


def _tec_body(heads_hbm, rel_hbm, tails_hbm, entity_hbm, relw_hbm, out_hbm,
              idx_h, idx_r, idx_t, rel_sh, hrbufs, tbufs, obufs,
              hsems, tsems, rsems, osems):
    batch = out_hbm.shape[0]
    embed = out_hbm.shape[1]
    b_per_w = batch // NUM_WORKERS
    n_chunks = b_per_w // CHUNK
    groups = n_chunks // N_SLOT
    sid = lax.axis_index("s")
    wid = sid * NUM_CORES + lax.axis_index("c")
    base = wid * b_per_w

    pltpu.sync_copy(heads_hbm.at[pl.ds(base, b_per_w)], idx_h)
    pltpu.sync_copy(rel_hbm.at[pl.ds(base, b_per_w)], idx_r)
    pltpu.sync_copy(tails_hbm.at[pl.ds(base, b_per_w)], idx_t)

    def h_copy(c, k):
        s = pl.ds(c * CHUNK, CHUNK)
        return pltpu.make_async_copy(entity_hbm.at[idx_h.at[s]], hrbufs[k],
                                     hsems[k])

    def t_copy(c, k):
        s = pl.ds(c * CHUNK, CHUNK)
        return pltpu.make_async_copy(entity_hbm.at[idx_t.at[s]], tbufs[k],
                                     tsems[k])

    def r_add(c, k):
        s = pl.ds(c * CHUNK, CHUNK)
        return pltpu.async_copy(rel_sh.at[idx_r.at[s]], hrbufs[k], rsems[k],
                                add=True)

    def r_wait(k):
        pltpu.make_async_copy(rel_sh.at[pl.ds(0, CHUNK)], hrbufs[k],
                              rsems[k]).wait()

    def o_copy(c, k):
        return pltpu.make_async_copy(
            obufs[k], out_hbm.at[pl.ds(base + c * CHUNK, CHUNK)], osems[k])

    for k in range(N_SLOT):
        h_copy(k, k).start()
        t_copy(k, k).start()

    @pl.when(sid == 0)
    def _stage_rel():
        pltpu.sync_copy(relw_hbm, rel_sh)
    plsc.subcore_barrier()

    h_copy(0, 0).wait()
    r_add(0, 0)

    def group_body(g, carry):
        for k in range(N_SLOT):
            c = g * N_SLOT + k
            kn = (k + 1) % N_SLOT

            @pl.when(c + 1 < n_chunks)
            def _prime_next_radd():
                h_copy(c + 1, kn).wait()
                r_add(c + 1, kn)

            r_wait(k)
            t_copy(c, k).wait()

            @pl.when(g > 0)
            def _reclaim_obuf():
                o_copy(c, k).wait()

            hr = hrbufs[k]
            t = tbufs[k]
            o = obufs[k]

            @plsc.parallel_loop(0, CHUNK, 1, unroll=4)
            def _compute(j):
                for kk in range(embed // LANES):
                    s = pl.ds(kk * LANES, LANES)
                    o[j, s] = hr[j, s] - t[j, s]

            o_copy(c, k).start()

            @pl.when(g + 1 < groups)
            def _prefetch():
                h_copy(c + N_SLOT, k).start()
                t_copy(c + N_SLOT, k).start()
        return carry

    lax.fori_loop(0, groups, group_body, 0)

    for k in range(N_SLOT):
        o_copy(n_chunks - N_SLOT + k, k).wait()


def _body_wrapper(heads_hbm, rel_hbm, tails_hbm, entity_hbm, relw_hbm,
                  out_hbm, idx_h, idx_r, idx_t, *rest):
    n = N_SLOT
    rel_sh = rest[-1]
    rest = rest[:-1]
    hrbufs = tuple(rest[0:n])
    tbufs = tuple(rest[n:2 * n])
    obufs = tuple(rest[2 * n:3 * n])
    hsems = tuple(rest[3 * n:4 * n])
    tsems = tuple(rest[4 * n:5 * n])
    rsems = tuple(rest[5 * n:6 * n])
    osems = tuple(rest[6 * n:7 * n])
    _tec_body(heads_hbm, rel_hbm, tails_hbm, entity_hbm, relw_hbm, out_hbm,
              idx_h, idx_r, idx_t, rel_sh, hrbufs, tbufs, obufs,
              hsems, tsems, rsems, osems)


def kernel(heads, relations, tails, entity_weight, rel_weight):
    batch = heads.shape[0]
    embed = entity_weight.shape[1]
    b_per_w = batch // NUM_WORKERS
    heads = heads.astype(jnp.int32)
    relations = relations.astype(jnp.int32)
    tails = tails.astype(jnp.int32)

    mesh = plsc.VectorSubcoreMesh(core_axis_name="c", subcore_axis_name="s")
    scratch = [pltpu.VMEM((b_per_w,), jnp.int32)] * 3
    scratch += [pltpu.VMEM((CHUNK, embed), jnp.float32)] * (3 * N_SLOT)
    scratch += [pltpu.SemaphoreType.DMA] * (4 * N_SLOT)
    scratch += [pltpu.VMEM_SHARED(rel_weight.shape, jnp.float32)]
    run = functools.partial(
        pl.kernel,
        mesh=mesh,
        out_type=jax.ShapeDtypeStruct((batch, embed), jnp.float32),
        scratch_types=scratch,
    )(_body_wrapper)
    return run(heads, relations, tails, entity_weight, rel_weight)


# speedup vs baseline: 1.1668x; 1.0438x over previous
import functools

import jax
import jax.numpy as jnp
from jax import lax
from jax.experimental import pallas as pl
from jax.experimental.pallas import tpu as pltpu
from jax.experimental.pallas import tpu_sc as plsc

LANES = 16
NUM_CORES = 2
NUM_SUBCORES = 16
NUM_WORKERS = NUM_CORES * NUM_SUBCORES
CHUNK = 64
N_SLOT = 4


def _tec_body(heads_hbm, rel_hbm, tails_hbm, entity_hbm, relw_hbm, out_hbm,
              idx_h, idx_r, idx_t, rel_sh, hrbufs, tbufs, obufs,
              hsems, tsems, rsems, osems):
    batch = out_hbm.shape[0]
    embed = out_hbm.shape[1]
    b_per_w = batch // NUM_WORKERS
    n_chunks = b_per_w // CHUNK
    groups = n_chunks // N_SLOT
    sid = lax.axis_index("s")
    wid = sid * NUM_CORES + lax.axis_index("c")
    base = wid * b_per_w

    ih = pltpu.make_async_copy(heads_hbm.at[pl.ds(base, b_per_w)], idx_h,
                               hsems[0])
    ir = pltpu.make_async_copy(rel_hbm.at[pl.ds(base, b_per_w)], idx_r,
                               rsems[0])
    it = pltpu.make_async_copy(tails_hbm.at[pl.ds(base, b_per_w)], idx_t,
                               tsems[0])
    ih.start()
    ir.start()
    it.start()
    ih.wait()
    it.wait()

    def h_copy(c, k):
        s = pl.ds(c * CHUNK, CHUNK)
        return pltpu.make_async_copy(entity_hbm.at[idx_h.at[s]], hrbufs[k],
                                     hsems[k])

    def t_copy(c, k):
        s = pl.ds(c * CHUNK, CHUNK)
        return pltpu.make_async_copy(entity_hbm.at[idx_t.at[s]], tbufs[k],
                                     tsems[k])

    def r_add(c, k):
        s = pl.ds(c * CHUNK, CHUNK)
        return pltpu.async_copy(rel_sh.at[idx_r.at[s]], hrbufs[k], rsems[k],
                                add=True)

    def r_wait(k):
        pltpu.make_async_copy(rel_sh.at[pl.ds(0, CHUNK)], hrbufs[k],
                              rsems[k]).wait()

    def o_copy(c, k):
        return pltpu.make_async_copy(
            obufs[k], out_hbm.at[pl.ds(base + c * CHUNK, CHUNK)], osems[k])

    for k in range(N_SLOT):
        h_copy(k, k).start()
        t_copy(k, k).start()

    @pl.when(sid == 0)
    def _stage_rel():
        pltpu.sync_copy(relw_hbm, rel_sh)
    plsc.subcore_barrier()

    ir.wait()
    h_copy(0, 0).wait()
    r_add(0, 0)

    def group_body(g, carry):
        for k in range(N_SLOT):
            c = g * N_SLOT + k
            kn = (k + 1) % N_SLOT

            @pl.when(c + 1 < n_chunks)
            def _prime_next_radd():
                h_copy(c + 1, kn).wait()
                r_add(c + 1, kn)

            r_wait(k)
            t_copy(c, k).wait()

            @pl.when(g > 0)
            def _reclaim_obuf():
                o_copy(c, k).wait()

            hr = hrbufs[k]
            t = tbufs[k]
            o = obufs[k]

            @plsc.parallel_loop(0, CHUNK, 1, unroll=4)
            def _compute(j):
                for kk in range(embed // LANES):
                    s = pl.ds(kk * LANES, LANES)
                    o[j, s] = hr[j, s] - t[j, s]

            o_copy(c, k).start()

            @pl.when(g + 1 < groups)
            def _prefetch():
                h_copy(c + N_SLOT, k).start()
                t_copy(c + N_SLOT, k).start()
        return carry

    lax.fori_loop(0, groups, group_body, 0)

    for k in range(N_SLOT):
        o_copy(n_chunks - N_SLOT + k, k).wait()


def _body_wrapper(heads_hbm, rel_hbm, tails_hbm, entity_hbm, relw_hbm,
                  out_hbm, idx_h, idx_r, idx_t, *rest):
    n = N_SLOT
    rel_sh = rest[-1]
    rest = rest[:-1]
    hrbufs = tuple(rest[0:n])
    tbufs = tuple(rest[n:2 * n])
    obufs = tuple(rest[2 * n:3 * n])
    hsems = tuple(rest[3 * n:4 * n])
    tsems = tuple(rest[4 * n:5 * n])
    rsems = tuple(rest[5 * n:6 * n])
    osems = tuple(rest[6 * n:7 * n])
    _tec_body(heads_hbm, rel_hbm, tails_hbm, entity_hbm, relw_hbm, out_hbm,
              idx_h, idx_r, idx_t, rel_sh, hrbufs, tbufs, obufs,
              hsems, tsems, rsems, osems)


def kernel(heads, relations, tails, entity_weight, rel_weight):
    batch = heads.shape[0]
    embed = entity_weight.shape[1]
    b_per_w = batch // NUM_WORKERS
    heads = heads.astype(jnp.int32)
    relations = relations.astype(jnp.int32)
    tails = tails.astype(jnp.int32)

    mesh = plsc.VectorSubcoreMesh(core_axis_name="c", subcore_axis_name="s")
    scratch = [pltpu.VMEM((b_per_w,), jnp.int32)] * 3
    scratch += [pltpu.VMEM((CHUNK, embed), jnp.float32)] * (3 * N_SLOT)
    scratch += [pltpu.SemaphoreType.DMA] * (4 * N_SLOT)
    scratch += [pltpu.VMEM_SHARED(rel_weight.shape, jnp.float32)]
    run = functools.partial(
        pl.kernel,
        mesh=mesh,
        out_type=jax.ShapeDtypeStruct((batch, embed), jnp.float32),
        scratch_types=scratch,
    )(_body_wrapper)
    return run(heads, relations, tails, entity_weight, rel_weight)
